# Initial kernel scaffold; baseline (speedup 1.0000x reference)
import jax
import jax.numpy as jnp
from jax.experimental import pallas as pl


def kernel(node_emb, edge_index, edge_weight, cas_idx, user_idx):
    raise NotImplementedError("write your pallas kernel here")


# speedup vs baseline: 3.2796x; 3.2796x over previous
import functools

import jax
import jax.numpy as jnp
from jax import lax
from jax.experimental import pallas as pl
from jax.experimental.pallas import tpu as pltpu
from jax.experimental.pallas import tpu_sc as plsc

N_CAS = 2000
N_USER = 8000
N = N_CAS + N_USER
E = 320000
D = 128
B = 4096
LAYERS = 3

NC = 2
NS = 16
NW = NC * NS
C = 128
N_PAD = 10240
ROWS_PER_SUB = N_PAD // NS
ROWS_PER_W = N_PAD // NW
NCHUNK = (E + NW * C - 1) // (NW * C)
EPW = NCHUNK * C
E_PAD = EPW * NW
OUT_ROWS_PER_W = 2 * B // NW
DSEG = D // 16

_mesh = plsc.VectorSubcoreMesh(core_axis_name="c", subcore_axis_name="s")


def _wid():
    return lax.axis_index("c") * NS + lax.axis_index("s")


@functools.partial(
    pl.kernel,
    out_type=(
        jax.ShapeDtypeStruct((N_PAD, D), jnp.float32),
        jax.ShapeDtypeStruct((N_PAD, D), jnp.float32),
    ),
    mesh=_mesh,
    scratch_types=[
        pltpu.VMEM_SHARED((N_PAD, D), jnp.float32),
        pltpu.VMEM((C,), jnp.int32),
        pltpu.VMEM((C,), jnp.int32),
        pltpu.VMEM((C,), jnp.float32),
        pltpu.VMEM((C, D), jnp.float32),
        pltpu.SemaphoreType.DMA,
    ],
)
def _edge_layer(e_hbm, src_hbm, dst_hbm, w_hbm, zeros_hbm,
                p0_out, p1_out, acc, idxs, idxd, wv, rows, sem):
    c = lax.axis_index("c")
    s = lax.axis_index("s")
    wid = c * NS + s
    r0 = s * ROWS_PER_SUB
    pltpu.sync_copy(zeros_hbm.at[pl.ds(r0, ROWS_PER_SUB)],
                    acc.at[pl.ds(r0, ROWS_PER_SUB)])
    plsc.subcore_barrier()

    base = wid * EPW

    def chunk(k, carry):
        off = base + k * C
        pltpu.sync_copy(src_hbm.at[pl.ds(off, C)], idxs)
        pltpu.sync_copy(dst_hbm.at[pl.ds(off, C)], idxd)
        pltpu.sync_copy(w_hbm.at[pl.ds(off, C)], wv)
        pltpu.async_copy(e_hbm.at[idxs], rows, sem).wait()

        def grp(gi, carry2):
            w16 = wv[pl.ds(gi * 16, 16)]
            base_row = gi * 16
            for l in range(16):
                w = w16[l]
                for j in range(DSEG):
                    sl = pl.ds(j * 16, 16)
                    rows[base_row + l, sl] = rows[base_row + l, sl] * w
            return carry2

        lax.fori_loop(0, C // 16, grp, 0)
        pltpu.sync_copy(rows, acc.at[idxd], add=True)
        return carry

    lax.fori_loop(0, NCHUNK, chunk, 0)
    plsc.subcore_barrier()

    @pl.when(c == 0)
    def _():
        pltpu.sync_copy(acc.at[pl.ds(r0, ROWS_PER_SUB)],
                        p0_out.at[pl.ds(r0, ROWS_PER_SUB)])

    @pl.when(c == 1)
    def _():
        pltpu.sync_copy(acc.at[pl.ds(r0, ROWS_PER_SUB)],
                        p1_out.at[pl.ds(r0, ROWS_PER_SUB)])


@functools.partial(
    pl.kernel,
    out_type=(
        jax.ShapeDtypeStruct((N_PAD, D), jnp.float32),
        jax.ShapeDtypeStruct((N_PAD, D), jnp.float32),
    ),
    mesh=_mesh,
    scratch_types=[
        pltpu.VMEM((ROWS_PER_W, D), jnp.float32),
        pltpu.VMEM((ROWS_PER_W, D), jnp.float32),
        pltpu.VMEM((ROWS_PER_W, D), jnp.float32),
    ],
)
def _combine(p0_hbm, p1_hbm, s_hbm, e_out, s_out, p0v, p1v, sv):
    wid = _wid()
    r0 = wid * ROWS_PER_W
    pltpu.sync_copy(p0_hbm.at[pl.ds(r0, ROWS_PER_W)], p0v)
    pltpu.sync_copy(p1_hbm.at[pl.ds(r0, ROWS_PER_W)], p1v)
    pltpu.sync_copy(s_hbm.at[pl.ds(r0, ROWS_PER_W)], sv)

    def row(i, carry):
        for j in range(DSEG):
            sl = pl.ds(j * 16, 16)
            e = p0v[i, sl] + p1v[i, sl]
            p0v[i, sl] = e
            sv[i, sl] = sv[i, sl] + e
        return carry

    lax.fori_loop(0, ROWS_PER_W, row, 0)
    pltpu.sync_copy(p0v, e_out.at[pl.ds(r0, ROWS_PER_W)])
    pltpu.sync_copy(sv, s_out.at[pl.ds(r0, ROWS_PER_W)])


@functools.partial(
    pl.kernel,
    out_type=jax.ShapeDtypeStruct((2 * B, D), jnp.float32),
    mesh=_mesh,
    scratch_types=[
        pltpu.VMEM((C,), jnp.int32),
        pltpu.VMEM((C, D), jnp.float32),
        pltpu.VMEM((C, D), jnp.float32),
        pltpu.VMEM((C, D), jnp.float32),
        pltpu.SemaphoreType.DMA,
    ],
)
def _final_gather(p0_hbm, p1_hbm, s_hbm, g_hbm, out_hbm,
                  gidx, rsv, r0v, r1v, sem):
    wid = _wid()
    for t in range(OUT_ROWS_PER_W // C):
        off = wid * OUT_ROWS_PER_W + t * C
        pltpu.sync_copy(g_hbm.at[pl.ds(off, C)], gidx)
        pltpu.async_copy(s_hbm.at[gidx], rsv, sem).wait()
        pltpu.async_copy(p0_hbm.at[gidx], r0v, sem).wait()
        pltpu.async_copy(p1_hbm.at[gidx], r1v, sem).wait()

        def row(i, carry):
            for j in range(DSEG):
                sl = pl.ds(j * 16, 16)
                rsv[i, sl] = (rsv[i, sl] + r0v[i, sl] + r1v[i, sl]) * 0.25
            return carry

        lax.fori_loop(0, C, row, 0)
        pltpu.sync_copy(rsv, out_hbm.at[pl.ds(off, C)])


def kernel(node_emb, edge_index, edge_weight, cas_idx, user_idx):
    src = edge_index[0]
    dst = edge_index[1]
    pad = E_PAD - E
    srcp = jnp.concatenate([src, jnp.zeros((pad,), jnp.int32)])
    dstp = jnp.concatenate([dst, jnp.zeros((pad,), jnp.int32)])
    wp = jnp.concatenate([edge_weight, jnp.zeros((pad,), jnp.float32)])
    e0 = jnp.zeros((N_PAD, D), jnp.float32).at[:N].set(node_emb)
    zeros = jnp.zeros((N_PAD, D), jnp.float32)
    g = jnp.concatenate([cas_idx, user_idx + N_CAS]).astype(jnp.int32)

    p0, p1 = _edge_layer(e0, srcp, dstp, wp, zeros)
    e1, s1 = _combine(p0, p1, e0)
    p0, p1 = _edge_layer(e1, srcp, dstp, wp, zeros)
    e2, s2 = _combine(p0, p1, s1)
    p0, p1 = _edge_layer(e2, srcp, dstp, wp, zeros)
    return _final_gather(p0, p1, s2, g)
